# Initial kernel scaffold; baseline (speedup 1.0000x reference)
import jax
import jax.numpy as jnp
from jax.experimental import pallas as pl


def kernel(memory, decoder_inputs, memory_lengths, pre_W1, pre_b1, pre_W2, pre_b2, arnn_Wih, arnn_Whh, arnn_bih, arnn_bhh, q_W, q_b, m_W, m_b, v_W, v_b, lc_W, lc_b, ld_W, ld_b, drnn_Wih, drnn_Whh, drnn_bih, drnn_bhh, proj_W, proj_b, gate_W, gate_b):
    raise NotImplementedError("write your pallas kernel here")


# speedup vs baseline: 3.9982x; 3.9982x over previous
import functools

import jax
import jax.numpy as jnp
from jax.experimental import pallas as pl
from jax.experimental.pallas import tpu as pltpu

B, T_ENC, T_DEC = 16, 128, 200
MEL, EMB, RNN, PRE, ATT, NF, KS = 80, 512, 1024, 256, 128, 32, 31
PAD = (KS - 1) // 2
NBLK = 4
TBLK = T_ENC // NBLK
W0S = (0, 16, 48, 64)
WW = 64

F32 = jnp.float32
BF16 = jnp.bfloat16


def _prenet_body(x_ref, m12_ref, w1t_ref, b1_ref, w2t_ref, b2_ref, h_ref):
    f32dot = functools.partial(jnp.dot, preferred_element_type=F32)
    m12 = m12_ref[...].astype(F32)
    h1 = jax.nn.relu(f32dot(x_ref[...], w1t_ref[...]) + b1_ref[...]) * m12[:, :PRE]
    h_ref[...] = jax.nn.relu(f32dot(h1, w2t_ref[...]) + b2_ref[...]) * m12[:, PRE:]


def _decoder_body(h_ref,
                  awx_ref, awc_ref, awhh_ref, ab_ref,
                  dwh_ref, dwc_ref, dwhh_ref, db_ref,
                  qwt_ref, gband_ref, vb_ref, pm_ref, maskneg_ref,
                  mem_ref, bd_ref, wp1_ref, wp2_ref, pgb_ref,
                  out_mel_ref, out_al_ref):
    f32dot = functools.partial(jnp.dot, preferred_element_type=F32)

    def step(t, carry):
        ah, ac, dh, dc, aw, awc, ctx = carry

        h_t = h_ref[pl.ds(t, 1)][0]

        ga = (f32dot(h_t.astype(BF16), awx_ref[...])
              + f32dot(ctx.astype(BF16), awc_ref[...])
              + f32dot(ah.astype(BF16), awhh_ref[...])
              + ab_ref[...])
        ii = jax.nn.sigmoid(ga[:, :RNN])
        ff = jax.nn.sigmoid(ga[:, RNN:2 * RNN])
        gg = jnp.tanh(ga[:, 2 * RNN:3 * RNN])
        oo = jax.nn.sigmoid(ga[:, 3 * RNN:])
        ac = ff * ac + ii * gg
        ah = oo * jnp.tanh(ac)
        ah16 = ah.astype(BF16)

        gd_p = (f32dot(ah16, dwh_ref[...])
                + f32dot(dh.astype(BF16), dwhh_ref[...])
                + db_ref[...])

        pq = f32dot(ah16, qwt_ref[...])
        pqr = pltpu.repeat(pq, TBLK, axis=1)
        aw16 = aw.astype(BF16)
        awc16 = awc.astype(BF16)
        e_parts = []
        for k in range(NBLK):
            w0 = W0S[k]
            cat_k = jnp.concatenate(
                [aw16[:, w0:w0 + WW], awc16[:, w0:w0 + WW]], axis=1)
            loc_k = f32dot(cat_k, gband_ref[k])
            s_k = jnp.tanh(pqr + loc_k + pm_ref[:, k * TBLK * ATT:(k + 1) * TBLK * ATT])
            e_parts.append(f32dot(s_k.astype(BF16), vb_ref[k]))
        e = jnp.concatenate(e_parts, axis=1) + maskneg_ref[...]

        p = jnp.exp(e)
        aw = p / jnp.sum(p, axis=1, keepdims=True)

        awt = pltpu.repeat(aw, B, axis=1) * bd_ref[...]
        ctx = f32dot(awt, mem_ref[...])
        awc = awc + aw

        ctx16 = ctx.astype(BF16)
        gd = gd_p + f32dot(ctx16, dwc_ref[...])
        ii = jax.nn.sigmoid(gd[:, :RNN])
        ff = jax.nn.sigmoid(gd[:, RNN:2 * RNN])
        gg = jnp.tanh(gd[:, 2 * RNN:3 * RNN])
        oo = jax.nn.sigmoid(gd[:, 3 * RNN:])
        dc = ff * dc + ii * gg
        dh = oo * jnp.tanh(dc)

        mel = f32dot(dh, wp1_ref[...]) + f32dot(ctx, wp2_ref[...]) + pgb_ref[...]
        out_mel_ref[pl.ds(t, 1)] = mel[None]
        out_al_ref[pl.ds(t, 1)] = aw[None]
        return ah, ac, dh, dc, aw, awc, ctx

    z = lambda *s: jnp.zeros(s, F32)
    init = (z(B, RNN), z(B, RNN), z(B, RNN), z(B, RNN),
            z(B, T_ENC), z(B, T_ENC), z(B, EMB))
    jax.lax.fori_loop(0, T_DEC, step, init)


def kernel(memory, decoder_inputs, memory_lengths, pre_W1, pre_b1, pre_W2, pre_b2,
           arnn_Wih, arnn_Whh, arnn_bih, arnn_bhh, q_W, q_b, m_W, m_b, v_W, v_b,
           lc_W, lc_b, ld_W, ld_b, drnn_Wih, drnn_Whh, drnn_bih, drnn_bhh,
           proj_W, proj_b, gate_W, gate_b):
    x = decoder_inputs.transpose(2, 0, 1)
    x = jnp.concatenate([jnp.zeros((1, B, MEL), F32), x[:-1]], axis=0)
    x = jnp.pad(x, ((0, 0), (0, 0), (0, T_ENC - MEL)))
    x2d = x.reshape(T_DEC * B, T_ENC)

    dk = jax.random.split(jax.random.key(42), 2)
    m1 = jax.random.bernoulli(dk[0], 0.5, (T_DEC + 1, B, PRE))[:T_DEC]
    m2 = jax.random.bernoulli(dk[1], 0.5, (T_DEC + 1, B, PRE))[:T_DEC]
    m12 = (jnp.concatenate([m1, m2], axis=2).astype(jnp.int8) * 2
           ).reshape(T_DEC * B, 2 * PRE)

    w1t = jnp.pad(pre_W1.T, ((0, T_ENC - MEL), (0, 0)))
    b1 = pre_b1[None]
    w2t = pre_W2.T
    b2 = pre_b2[None]

    awx = arnn_Wih[:, :PRE].T.astype(BF16)
    awcw = arnn_Wih[:, PRE:].T.astype(BF16)
    awhh = arnn_Whh.T.astype(BF16)
    ab = (arnn_bih + arnn_bhh)[None]
    dwh = drnn_Wih[:, :RNN].T.astype(BF16)
    dwc = drnn_Wih[:, RNN:].T.astype(BF16)
    dwhh = drnn_Whh.T.astype(BF16)
    db = (drnn_bih + drnn_bhh)[None]
    qwt = q_W.T.astype(BF16)

    A = jnp.einsum('af,fcd->acd', ld_W, lc_W)
    tcol = jnp.arange(T_ENC)
    gband = []
    for k in range(NBLK):
        tau = W0S[k] + jnp.arange(WW)
        tk = tcol[k * TBLK:(k + 1) * TBLK]
        delta = tau[:, None] - tk[None, :] + PAD
        valid = (delta >= 0) & (delta < KS)
        dcl = jnp.clip(delta, 0, KS - 1)
        g = jnp.where(valid[None, :, :, None],
                      jnp.transpose(A, (1, 2, 0))[:, dcl, :], 0.0)
        gband.append(g.reshape(2 * WW, TBLK * ATT))
    gband = jnp.stack(gband).astype(BF16)

    bias_la = ld_W @ lc_b + ld_b
    pm = (jnp.einsum('bte,ae->bta', memory, m_W) + m_b + q_b + bias_la)
    pm = pm.reshape(B, T_ENC * ATT)

    v = v_W[0]
    vb = []
    for k in range(NBLK):
        vb.append((jnp.eye(TBLK)[:, None, :] * v[None, :, None]).reshape(TBLK * ATT, TBLK))
    vb = jnp.stack(vb).astype(BF16)

    maskneg = jnp.where(jnp.arange(T_ENC)[None, :] >= memory_lengths[:, None],
                        -1e30, 0.0).astype(F32)
    mem_stack = memory.reshape(B * T_ENC, EMB)
    bd = (jnp.arange(B)[:, None] == (jnp.arange(B * T_ENC) // T_ENC)[None, :]).astype(F32)

    wp1 = jnp.concatenate([proj_W[:, :RNN], gate_W[:, :RNN]], axis=0)
    wp2 = jnp.concatenate([proj_W[:, RNN:], gate_W[:, RNN:]], axis=0)
    wp1 = jnp.pad(wp1, ((0, T_ENC - MEL - 1), (0, 0))).T
    wp2 = jnp.pad(wp2, ((0, T_ENC - MEL - 1), (0, 0))).T
    pgb = jnp.pad(jnp.concatenate([proj_b, gate_b]), (0, T_ENC - MEL - 1))[None]

    vmem = functools.partial(pl.BlockSpec, memory_space=pltpu.VMEM)
    h_all = pl.pallas_call(
        _prenet_body,
        out_shape=jax.ShapeDtypeStruct((T_DEC * B, PRE), F32),
        in_specs=[vmem()] * 6,
        out_specs=vmem(),
        compiler_params=pltpu.CompilerParams(
            vmem_limit_bytes=60000 * 1024,
        ),
    )(x2d, m12, w1t, b1, w2t, b2)
    h3d = h_all.reshape(T_DEC, B, PRE)

    out_mel, out_al = pl.pallas_call(
        _decoder_body,
        out_shape=(jax.ShapeDtypeStruct((T_DEC, B, T_ENC), F32),
                   jax.ShapeDtypeStruct((T_DEC, B, T_ENC), F32)),
        in_specs=[vmem()] * 19,
        out_specs=(vmem(), vmem()),
        compiler_params=pltpu.CompilerParams(
            vmem_limit_bytes=60000 * 1024,
        ),
    )(h3d, awx, awcw, awhh, ab,
      dwh, dwc, dwhh, db, qwt, gband, vb, pm, maskneg,
      mem_stack, bd, wp1, wp2, pgb)

    mel_outputs = out_mel[:, :, :MEL].transpose(1, 2, 0)
    gate_outputs = out_mel[:, :, MEL].T
    alignments = out_al.transpose(1, 0, 2)
    return mel_outputs, gate_outputs, alignments
